# baseline (speedup 1.0000x reference)
import functools

import jax
import jax.numpy as jnp
from jax import lax
from jax.experimental import pallas as pl
from jax.experimental.pallas import tpu as pltpu
from jax.experimental.pallas import tpu_sc as plsc

B = 4096
D_NUM = 13
N_CAT = 26
PER_FIELD = 1000
D = 128
N_TOK = 1 + D_NUM + N_CAT

NW = 32
BPW = B // NW
C = 8
NCHUNK = BPW // C


def _fold_bias_body(t_ref, b_ref, o_ref):
    m = pl.program_id(0)
    for h in range(2):
        o_ref[pl.ds(h * PER_FIELD, PER_FIELD), :] = (
            t_ref[pl.ds(h * PER_FIELD, PER_FIELD), :]
            + b_ref[pl.ds(D_NUM + 2 * m + h, 1), :])


def _fold_bias(category_embeddings, bias):
    return pl.pallas_call(
        _fold_bias_body,
        grid=(N_CAT // 2,),
        in_specs=[
            pl.BlockSpec((2 * PER_FIELD, D), lambda m: (m, 0)),
            pl.BlockSpec((D_NUM + N_CAT, D), lambda m: (0, 0)),
        ],
        out_specs=pl.BlockSpec((2 * PER_FIELD, D), lambda m: (m, 0)),
        out_shape=jax.ShapeDtypeStruct((N_CAT * PER_FIELD, D), jnp.float32),
    )(category_embeddings, bias)


def _sc_body(xn_hbm, idx_hbm, w_hbm, bn_hbm, table_hbm, out_hbm,
             w_v, bn_v, xn_v, idx_v, slab,
             sem_g0, sem_g1, sem_o0, sem_o1, sem_i0, sem_i1):
    wid = lax.axis_index("s") * 2 + lax.axis_index("c")
    base = wid * BPW
    sem_g = (sem_g0, sem_g1)
    sem_o = (sem_o0, sem_o1)
    sem_i = (sem_i0, sem_i1)

    pltpu.sync_copy(w_hbm, w_v)
    pltpu.sync_copy(bn_hbm, bn_v)

    def fire_gathers(q, i):
        for c in range(C):
            pltpu.async_copy(
                table_hbm.at[idx_v.at[q, c, pl.ds(0, N_CAT)]],
                slab.at[q, c, pl.ds(1 + D_NUM, N_CAT)],
                sem_g[q])

    def wait_gathers(q):
        for c in range(C):
            pltpu.make_async_copy(
                table_hbm.at[idx_v.at[q, c, pl.ds(0, N_CAT)]],
                slab.at[q, c, pl.ds(1 + D_NUM, N_CAT)],
                sem_g[q]).wait()

    def prefetch_in(q, b):
        pltpu.async_copy(xn_hbm.at[pl.ds(b, C)], xn_v.at[q], sem_i[q])
        pltpu.async_copy(idx_hbm.at[pl.ds(b, C)], idx_v.at[q], sem_i[q])

    def wait_in(q, b):
        pltpu.make_async_copy(
            xn_hbm.at[pl.ds(b, C)], xn_v.at[q], sem_i[q]).wait()
        pltpu.make_async_copy(
            idx_hbm.at[pl.ds(b, C)], idx_v.at[q], sem_i[q]).wait()

    prefetch_in(0, base)
    prefetch_in(1, base + C)
    wait_in(0, base)
    fire_gathers(0, 0)

    def pair_body(ip, carry):
        for p in range(2):
            i = ip * 2 + p
            b0 = base + i * C

            wait_gathers(p)

            xnrows = [xn_v[p, c] for c in range(C)]
            for j in range(1 + D_NUM):
                w16 = [w_v[j, pl.ds(r * 16, 16)] for r in range(D // 16)]
                b16 = [bn_v[j, pl.ds(r * 16, 16)] for r in range(D // 16)]
                for c in range(C):
                    sv = jnp.full((16,), xnrows[c][j], jnp.float32)
                    for r in range(D // 16):
                        slab[p, c, j, pl.ds(r * 16, 16)] = w16[r] * sv + b16[r]

            pltpu.async_copy(slab.at[p], out_hbm.at[pl.ds(b0, C)], sem_o[p])

            bn1 = lax.min(b0 + C, B - C)
            wait_in(1 - p, bn1)

            bn2 = lax.min(b0 + 2 * C, B - C)
            prefetch_in(p, bn2)

            if p == 0:
                @pl.when(ip > 0)
                def _wait_out_prev():
                    pltpu.make_async_copy(
                        slab.at[1], out_hbm.at[pl.ds(b0, C)],
                        sem_o[1]).wait()
                fire_gathers(1, i + 1)
            else:
                pltpu.make_async_copy(
                    slab.at[0], out_hbm.at[pl.ds(b0, C)], sem_o[0]).wait()

                @pl.when(ip < NCHUNK // 2 - 1)
                def _fire_next():
                    fire_gathers(0, i + 1)
        return carry

    lax.fori_loop(0, NCHUNK // 2, pair_body, 0)

    pltpu.make_async_copy(
        slab.at[1], out_hbm.at[pl.ds(base + (NCHUNK - 1) * C, C)],
        sem_o[1]).wait()
    wait_in(1, base)


@jax.jit
def kernel(x_num, x_cat, weight, category_embeddings, bias, category_offsets):
    n = x_num.shape[0]
    xn = jnp.concatenate(
        [jnp.ones((n, 1), jnp.float32), x_num, jnp.zeros((n, 2), jnp.float32)],
        axis=1)
    idx = (x_cat.astype(jnp.int32)
           + category_offsets.astype(jnp.int32)[None, :])
    idx = jnp.pad(idx, ((0, 0), (0, 32 - N_CAT)))
    bias_num = jnp.concatenate(
        [jnp.zeros((1, D), jnp.float32), bias[:D_NUM]], axis=0)

    mesh = plsc.VectorSubcoreMesh(core_axis_name="c", subcore_axis_name="s")
    sc = functools.partial(
        pl.kernel,
        out_type=jax.ShapeDtypeStruct((B, N_TOK, D), jnp.float32),
        mesh=mesh,
        scratch_types=[
            pltpu.VMEM((1 + D_NUM, D), jnp.float32),
            pltpu.VMEM((1 + D_NUM, D), jnp.float32),
            pltpu.VMEM((2, C, 16), jnp.float32),
            pltpu.VMEM((2, C, 32), jnp.int32),
            pltpu.VMEM((2, C, N_TOK, D), jnp.float32),
            pltpu.SemaphoreType.DMA,
            pltpu.SemaphoreType.DMA,
            pltpu.SemaphoreType.DMA,
            pltpu.SemaphoreType.DMA,
            pltpu.SemaphoreType.DMA,
            pltpu.SemaphoreType.DMA,
        ],
    )(_sc_body)
    table2 = _fold_bias(category_embeddings, bias)
    return sc(xn, idx, weight, bias_num, table2)


# speedup vs baseline: 1.0292x; 1.0292x over previous
import functools

import jax
import jax.numpy as jnp
from jax import lax
from jax.experimental import pallas as pl
from jax.experimental.pallas import tpu as pltpu
from jax.experimental.pallas import tpu_sc as plsc

B = 4096
D_NUM = 13
N_CAT = 26
PER_FIELD = 1000
D = 128
N_TOK = 1 + D_NUM + N_CAT

NW = 32
BPW = B // NW
C = 8
NCHUNK = BPW // C


def _prep_body(t_ref, b_ref, xnum_ref, xcat_ref, o_ref, xn_ref, idx_ref,
               bn_ref):
    m = pl.program_id(0)
    for h in range(2):
        o_ref[pl.ds(h * PER_FIELD, PER_FIELD), :] = (
            t_ref[pl.ds(h * PER_FIELD, PER_FIELD), :]
            + b_ref[pl.ds(D_NUM + 2 * m + h, 1), :])

    @pl.when(m == 0)
    def _prep_inputs():
        xn_ref[...] = jnp.concatenate(
            [jnp.ones((B, 1), jnp.float32), xnum_ref[...],
             jnp.zeros((B, 2), jnp.float32)], axis=1)
        offs = PER_FIELD * lax.broadcasted_iota(jnp.int32, (B, N_CAT), 1)
        idx_ref[...] = jnp.pad(
            xcat_ref[...] + offs, ((0, 0), (0, 32 - N_CAT)))
        bn_ref[...] = jnp.concatenate(
            [jnp.zeros((1, D), jnp.float32), b_ref[pl.ds(0, D_NUM), :]],
            axis=0)


def _prep(category_embeddings, bias, x_num, x_cat):
    return pl.pallas_call(
        _prep_body,
        grid=(N_CAT // 2,),
        in_specs=[
            pl.BlockSpec((2 * PER_FIELD, D), lambda m: (m, 0)),
            pl.BlockSpec((D_NUM + N_CAT, D), lambda m: (0, 0)),
            pl.BlockSpec((B, D_NUM), lambda m: (0, 0)),
            pl.BlockSpec((B, N_CAT), lambda m: (0, 0)),
        ],
        out_specs=[
            pl.BlockSpec((2 * PER_FIELD, D), lambda m: (m, 0)),
            pl.BlockSpec((B, 16), lambda m: (0, 0)),
            pl.BlockSpec((B, 32), lambda m: (0, 0)),
            pl.BlockSpec((1 + D_NUM, D), lambda m: (0, 0)),
        ],
        out_shape=[
            jax.ShapeDtypeStruct((N_CAT * PER_FIELD, D), jnp.float32),
            jax.ShapeDtypeStruct((B, 16), jnp.float32),
            jax.ShapeDtypeStruct((B, 32), jnp.int32),
            jax.ShapeDtypeStruct((1 + D_NUM, D), jnp.float32),
        ],
    )(category_embeddings, bias, x_num, x_cat)


def _sc_body(xn_hbm, idx_hbm, w_hbm, bn_hbm, table_hbm, out_hbm,
             w_v, bn_v, xn_v, idx_v, slab,
             sem_g0, sem_g1, sem_o0, sem_o1, sem_i0, sem_i1):
    wid = lax.axis_index("s") * 2 + lax.axis_index("c")
    base = wid * BPW
    sem_g = (sem_g0, sem_g1)
    sem_o = (sem_o0, sem_o1)
    sem_i = (sem_i0, sem_i1)

    pltpu.sync_copy(w_hbm, w_v)
    pltpu.sync_copy(bn_hbm, bn_v)

    pltpu.async_copy(xn_hbm.at[pl.ds(base, C)], xn_v.at[0], sem_i[0])
    pltpu.async_copy(idx_hbm.at[pl.ds(base, C)], idx_v.at[0], sem_i[0])

    def pair_body(ip, carry):
        for p in range(2):
            i = ip * 2 + p
            b0 = base + i * C

            pltpu.make_async_copy(
                xn_hbm.at[pl.ds(b0, C)], xn_v.at[p], sem_i[p]).wait()
            pltpu.make_async_copy(
                idx_hbm.at[pl.ds(b0, C)], idx_v.at[p], sem_i[p]).wait()

            bnx = lax.min(b0 + C, B - C)
            pltpu.async_copy(
                xn_hbm.at[pl.ds(bnx, C)], xn_v.at[1 - p], sem_i[1 - p])
            pltpu.async_copy(
                idx_hbm.at[pl.ds(bnx, C)], idx_v.at[1 - p], sem_i[1 - p])

            @pl.when(ip > 0)
            def _wait_prev():
                pltpu.make_async_copy(
                    slab.at[p], out_hbm.at[pl.ds(b0, C)], sem_o[p]).wait()

            copies = []
            for c in range(C):
                copies.append(pltpu.async_copy(
                    table_hbm.at[idx_v.at[p, c, pl.ds(0, N_CAT)]],
                    slab.at[p, c, pl.ds(1 + D_NUM, N_CAT)],
                    sem_g[p]))

            xnrows = [xn_v[p, c] for c in range(C)]
            for j in range(1 + D_NUM):
                w16 = [w_v[j, pl.ds(r * 16, 16)] for r in range(D // 16)]
                b16 = [bn_v[j, pl.ds(r * 16, 16)] for r in range(D // 16)]
                for c in range(C):
                    sv = jnp.full((16,), xnrows[c][j], jnp.float32)
                    for r in range(D // 16):
                        slab[p, c, j, pl.ds(r * 16, 16)] = w16[r] * sv + b16[r]

            for cp in copies:
                cp.wait()

            pltpu.async_copy(slab.at[p], out_hbm.at[pl.ds(b0, C)], sem_o[p])
        return carry

    lax.fori_loop(0, NCHUNK // 2, pair_body, 0)

    for p in range(2):
        b_last = base + (NCHUNK - 2 + p) * C
        pltpu.make_async_copy(
            slab.at[p], out_hbm.at[pl.ds(b_last, C)], sem_o[p]).wait()

    pltpu.make_async_copy(
        xn_hbm.at[pl.ds(base, C)], xn_v.at[0], sem_i[0]).wait()
    pltpu.make_async_copy(
        idx_hbm.at[pl.ds(base, C)], idx_v.at[0], sem_i[0]).wait()


@jax.jit
def kernel(x_num, x_cat, weight, category_embeddings, bias, category_offsets):
    del category_offsets
    mesh = plsc.VectorSubcoreMesh(core_axis_name="c", subcore_axis_name="s")
    sc = functools.partial(
        pl.kernel,
        out_type=jax.ShapeDtypeStruct((B, N_TOK, D), jnp.float32),
        mesh=mesh,
        scratch_types=[
            pltpu.VMEM((1 + D_NUM, D), jnp.float32),
            pltpu.VMEM((1 + D_NUM, D), jnp.float32),
            pltpu.VMEM((2, C, 16), jnp.float32),
            pltpu.VMEM((2, C, 32), jnp.int32),
            pltpu.VMEM((2, C, N_TOK, D), jnp.float32),
            pltpu.SemaphoreType.DMA,
            pltpu.SemaphoreType.DMA,
            pltpu.SemaphoreType.DMA,
            pltpu.SemaphoreType.DMA,
            pltpu.SemaphoreType.DMA,
            pltpu.SemaphoreType.DMA,
        ],
    )(_sc_body)
    table2, xn, idx, bias_num = _prep(
        category_embeddings, bias, x_num, x_cat.astype(jnp.int32))
    return sc(xn, idx, weight, bias_num, table2)


# speedup vs baseline: 1.0487x; 1.0189x over previous
import functools

import jax
import jax.numpy as jnp
from jax import lax
from jax.experimental import pallas as pl
from jax.experimental.pallas import tpu as pltpu
from jax.experimental.pallas import tpu_sc as plsc

B = 4096
D_NUM = 13
N_CAT = 26
PER_FIELD = 1000
D = 128
N_TOK = 1 + D_NUM + N_CAT

NW = 32
BPW = B // NW
C = 8
NCHUNK = BPW // C


def _fold_bias_body(t_ref, b_ref, o_ref):
    m = pl.program_id(0)
    for h in range(2):
        o_ref[pl.ds(h * PER_FIELD, PER_FIELD), :] = (
            t_ref[pl.ds(h * PER_FIELD, PER_FIELD), :]
            + b_ref[pl.ds(D_NUM + 2 * m + h, 1), :])


def _fold_bias(category_embeddings, bias):
    return pl.pallas_call(
        _fold_bias_body,
        grid=(N_CAT // 2,),
        in_specs=[
            pl.BlockSpec((2 * PER_FIELD, D), lambda m: (m, 0)),
            pl.BlockSpec((D_NUM + N_CAT, D), lambda m: (0, 0)),
        ],
        out_specs=pl.BlockSpec((2 * PER_FIELD, D), lambda m: (m, 0)),
        out_shape=jax.ShapeDtypeStruct((N_CAT * PER_FIELD, D), jnp.float32),
    )(category_embeddings, bias)


def _sc_body(xn_hbm, idx_hbm, w_hbm, bn_hbm, table_hbm, out_hbm,
             w_v, bn_v, xn_v, idx_v, slab,
             sem_g0, sem_g1, sem_o0, sem_o1, sem_i0, sem_i1):
    wid = lax.axis_index("s") * 2 + lax.axis_index("c")
    base = wid * BPW
    sem_g = (sem_g0, sem_g1)
    sem_o = (sem_o0, sem_o1)
    sem_i = (sem_i0, sem_i1)

    pltpu.sync_copy(w_hbm, w_v)
    pltpu.sync_copy(bn_hbm, bn_v)

    pltpu.async_copy(xn_hbm.at[pl.ds(base, C)], xn_v.at[0], sem_i[0])
    pltpu.async_copy(idx_hbm.at[pl.ds(base, C)], idx_v.at[0], sem_i[0])

    def pair_body(ip, carry):
        for p in range(2):
            i = ip * 2 + p
            b0 = base + i * C

            pltpu.make_async_copy(
                xn_hbm.at[pl.ds(b0, C)], xn_v.at[p], sem_i[p]).wait()
            pltpu.make_async_copy(
                idx_hbm.at[pl.ds(b0, C)], idx_v.at[p], sem_i[p]).wait()

            bnx = lax.min(b0 + C, B - C)
            pltpu.async_copy(
                xn_hbm.at[pl.ds(bnx, C)], xn_v.at[1 - p], sem_i[1 - p])
            pltpu.async_copy(
                idx_hbm.at[pl.ds(bnx, C)], idx_v.at[1 - p], sem_i[1 - p])

            @pl.when(ip > 0)
            def _wait_prev():
                pltpu.make_async_copy(
                    slab.at[p], out_hbm.at[pl.ds(b0, C)], sem_o[p]).wait()

            copies = []
            for c in range(C):
                copies.append(pltpu.async_copy(
                    table_hbm.at[idx_v.at[p, c, pl.ds(0, N_CAT)]],
                    slab.at[p, c, pl.ds(1 + D_NUM, N_CAT)],
                    sem_g[p]))

            xnrows = [xn_v[p, c] for c in range(C)]
            H = C // 2
            for h in range(2):
                for j in range(1 + D_NUM):
                    w16 = [w_v[j, pl.ds(r * 16, 16)] for r in range(D // 16)]
                    b16 = [bn_v[j, pl.ds(r * 16, 16)] for r in range(D // 16)]
                    for c in range(h * H, (h + 1) * H):
                        sv = jnp.full((16,), xnrows[c][j], jnp.float32)
                        for r in range(D // 16):
                            slab[p, c, j, pl.ds(r * 16, 16)] = (
                                w16[r] * sv + b16[r])
                for c in range(h * H, (h + 1) * H):
                    copies[c].wait()
                pltpu.async_copy(
                    slab.at[p, pl.ds(h * H, H)],
                    out_hbm.at[pl.ds(b0 + h * H, H)], sem_o[p])
        return carry

    lax.fori_loop(0, NCHUNK // 2, pair_body, 0)

    for p in range(2):
        b_last = base + (NCHUNK - 2 + p) * C
        pltpu.make_async_copy(
            slab.at[p], out_hbm.at[pl.ds(b_last, C)], sem_o[p]).wait()

    pltpu.make_async_copy(
        xn_hbm.at[pl.ds(base, C)], xn_v.at[0], sem_i[0]).wait()
    pltpu.make_async_copy(
        idx_hbm.at[pl.ds(base, C)], idx_v.at[0], sem_i[0]).wait()


@jax.jit
def kernel(x_num, x_cat, weight, category_embeddings, bias, category_offsets):
    n = x_num.shape[0]
    xn = jnp.concatenate(
        [jnp.ones((n, 1), jnp.float32), x_num, jnp.zeros((n, 2), jnp.float32)],
        axis=1)
    idx = (x_cat.astype(jnp.int32)
           + category_offsets.astype(jnp.int32)[None, :])
    idx = jnp.pad(idx, ((0, 0), (0, 32 - N_CAT)))
    bias_num = jnp.concatenate(
        [jnp.zeros((1, D), jnp.float32), bias[:D_NUM]], axis=0)

    mesh = plsc.VectorSubcoreMesh(core_axis_name="c", subcore_axis_name="s")
    sc = functools.partial(
        pl.kernel,
        out_type=jax.ShapeDtypeStruct((B, N_TOK, D), jnp.float32),
        mesh=mesh,
        scratch_types=[
            pltpu.VMEM((1 + D_NUM, D), jnp.float32),
            pltpu.VMEM((1 + D_NUM, D), jnp.float32),
            pltpu.VMEM((2, C, 16), jnp.float32),
            pltpu.VMEM((2, C, 32), jnp.int32),
            pltpu.VMEM((2, C, N_TOK, D), jnp.float32),
            pltpu.SemaphoreType.DMA,
            pltpu.SemaphoreType.DMA,
            pltpu.SemaphoreType.DMA,
            pltpu.SemaphoreType.DMA,
            pltpu.SemaphoreType.DMA,
            pltpu.SemaphoreType.DMA,
        ],
    )(_sc_body)
    table2 = _fold_bias(category_embeddings, bias)
    return sc(xn, idx, weight, bias_num, table2)
